# baseline (speedup 1.0000x reference)
import functools

import jax
import jax.numpy as jnp
from jax import lax
from jax.experimental import pallas as pl
from jax.experimental.pallas import tpu as pltpu
from jax.experimental.pallas import tpu_sc as plsc

_INFO = plsc.get_sparse_core_info()
_NC = _INFO.num_cores
_NS = _INFO.num_subcores
_NW = _NC * _NS
_NBUF = 4


@functools.lru_cache(maxsize=None)
def _build(bsz, hist, vocab, dim):
    assert bsz % (_NW * _NBUF) == 0
    rows_per_w = bsz // _NW
    n_groups = rows_per_w // _NBUF
    parts = []
    off = 0
    while off < hist:
        ln = min(128, hist - off)
        parts.append((off, ln))
        off += ln
    mesh = plsc.VectorSubcoreMesh(core_axis_name="c", subcore_axis_name="s")

    @functools.partial(
        pl.kernel,
        mesh=mesh,
        out_type=jax.ShapeDtypeStruct((bsz, hist, dim), jnp.float32),
        scratch_types=[
            pltpu.VMEM((rows_per_w, hist), jnp.int32),
            tuple(pltpu.VMEM((_NBUF, ln, dim), jnp.float32) for _, ln in parts),
            pltpu.SemaphoreType.DMA((_NBUF,)),
            pltpu.SemaphoreType.DMA((_NBUF,)),
        ],
        compiler_params=pltpu.CompilerParams(use_tc_tiling_on_sc=False),
    )
    def gather_kernel(x_hbm, table_hbm, out_hbm, idx_v, bufs, gsems, wsems):
        wid = lax.axis_index("s") * _NC + lax.axis_index("c")
        row0 = wid * rows_per_w
        pltpu.sync_copy(x_hbm.at[pl.ds(row0, rows_per_w)], idx_v)

        def start_gather(r, b):
            for p, (off, ln) in enumerate(parts):
                pltpu.make_async_copy(
                    table_hbm.at[idx_v.at[r].at[pl.ds(off, ln)]],
                    bufs[p].at[b],
                    gsems.at[b],
                ).start()

        def wait_gather(r, b):
            for p, (off, ln) in enumerate(parts):
                pltpu.make_async_copy(
                    table_hbm.at[idx_v.at[r].at[pl.ds(off, ln)]],
                    bufs[p].at[b],
                    gsems.at[b],
                ).wait()

        def start_write(r, b):
            for p, (off, ln) in enumerate(parts):
                pltpu.make_async_copy(
                    bufs[p].at[b],
                    out_hbm.at[row0 + r].at[pl.ds(off, ln)],
                    wsems.at[b],
                ).start()

        def wait_write(r, b):
            for p, (off, ln) in enumerate(parts):
                pltpu.make_async_copy(
                    bufs[p].at[b],
                    out_hbm.at[row0 + r].at[pl.ds(off, ln)],
                    wsems.at[b],
                ).wait()

        for b in range(_NBUF):
            start_gather(b, b)

        def group(g, carry):
            base = g * _NBUF
            for b in range(_NBUF):
                wait_gather(base + b, b)
                start_write(base + b, b)
            for b in range(_NBUF):
                wait_write(base + b, b)
                start_gather(base + _NBUF + b, b)
            return carry

        lax.fori_loop(0, n_groups - 1, group, 0)

        base = (n_groups - 1) * _NBUF
        for b in range(_NBUF):
            wait_gather(base + b, b)
            start_write(base + b, b)
        for b in range(_NBUF):
            wait_write(base + b, b)

    return gather_kernel


def kernel(x, table):
    bsz, hist = x.shape
    vocab, dim = table.shape
    return _build(bsz, hist, vocab, dim)(x.astype(jnp.int32), table)
